# baseline (speedup 1.0000x reference)
import functools

import jax
import jax.numpy as jnp
from jax import lax
from jax.experimental import pallas as pl
from jax.experimental.pallas import tpu as pltpu
from jax.experimental.pallas import tpu_sc as plsc

H, W, C = 64, 64, 3
F = H * W * C
B = 64
BUF = 50
SRC = BUF - 1 + B
NW = 32
CH = 5
NCHUNK = BUF // CH
I_PER_W = B // NW

_mesh = plsc.VectorSubcoreMesh(core_axis_name="c", subcore_axis_name="s")


@functools.partial(
    pl.kernel,
    mesh=_mesh,
    out_type=jax.ShapeDtypeStruct((B * BUF, F // 128, 128), jnp.float32),
    scratch_types=[
        pltpu.VMEM((2, CH, F // 128, 128), jnp.float32),
        pltpu.SemaphoreType.DMA,
        pltpu.SemaphoreType.DMA,
    ],
)
def _fifo_copy(src_hbm, out_hbm, vbuf, sem_out0, sem_out1):
    wid = lax.axis_index("s") * 2 + lax.axis_index("c")
    i0 = wid * I_PER_W
    out_sems = (sem_out0, sem_out1)
    pending = [None, None]
    for di in range(I_PER_W):
        i = i0 + di
        tbase = i * BUF
        for g in range(NCHUNK):
            b = (di * NCHUNK + g) & 1
            if pending[b] is not None:
                pending[b].wait()
            pltpu.sync_copy(src_hbm.at[pl.ds(i + g * CH, CH)], vbuf.at[b])
            pending[b] = pltpu.async_copy(
                vbuf.at[b], out_hbm.at[pl.ds(tbase + g * CH, CH)], out_sems[b])
    pending[0].wait()
    pending[1].wait()


def kernel(x, buffer):
    src = jnp.concatenate([buffer[1:], x], axis=0).reshape(SRC, F // 128, 128)
    out = _fifo_copy(src)
    return out.reshape(B, BUF, H, W, C)
